# baseline (speedup 1.0000x reference)
import functools

import jax
import jax.numpy as jnp
from jax.experimental import pallas as pl
from jax.experimental.pallas import tpu as pltpu

T = 64
D = 768
E = 64
I2 = 1536
CHUNK = 768
NCHUNK = I2 // CHUNK
LIMIT = 7.0
ALPHA = 1.702

_DN = (((1,), (1,)), ((), ()))


def _moe_body(x_ref, nw_ref, gw_ref, gb_ref, w1_ref, b1_ref, w2_ref, b2_ref,
              out_ref, t_ref, c_ref, s_ref):
    e = pl.program_id(0)
    c = pl.program_id(1)

    @pl.when((e == 0) & (c == 0))
    def _init():
        si = jax.lax.broadcasted_iota(jnp.int32, (CHUNK, CHUNK // 2), 0)
        sj = jax.lax.broadcasted_iota(jnp.int32, (CHUNK, CHUNK // 2), 1)
        s_ref[...] = (si == 2 * sj).astype(jnp.bfloat16)
        xv = x_ref[...]
        ms = jnp.mean(xv * xv, axis=1, keepdims=True)
        eps = jnp.finfo(jnp.float32).eps
        t = xv * jax.lax.rsqrt(ms + eps) * nw_ref[...]
        t_ref[...] = t
        g = jax.lax.dot_general(t, gw_ref[...], _DN,
                                preferred_element_type=jnp.float32)
        g = g + gb_ref[...]
        col = jax.lax.broadcasted_iota(jnp.int32, (T, E), 1)
        m1 = jnp.max(g, axis=1, keepdims=True)
        i1 = jnp.min(jnp.where(g == m1, col, E), axis=1, keepdims=True)
        oh1 = col == i1
        gm = jnp.where(oh1, -jnp.inf, g)
        m2 = jnp.max(gm, axis=1, keepdims=True)
        i2 = jnp.min(jnp.where(gm == m2, col, E), axis=1, keepdims=True)
        oh2 = col == i2
        p1 = jax.nn.sigmoid(m1 - m2)
        c_ref[...] = jnp.where(oh1, p1, 0.0) + jnp.where(oh2, 1.0 - p1, 0.0)
        out_ref[...] = xv

    t = t_ref[...].astype(jnp.bfloat16)
    b1 = b1_ref[0, :, pl.ds(c * CHUNK, CHUNK)]
    h = jax.lax.dot_general(t, w1_ref[0].astype(jnp.bfloat16), _DN,
                            preferred_element_type=jnp.float32) + b1
    glu = jnp.minimum(h, LIMIT)
    act = glu * jax.nn.sigmoid(ALPHA * glu)
    lin = jnp.clip(h, -LIMIT, LIMIT) + 1.0
    lin_shift = jnp.concatenate([lin[:, 1:], lin[:, :1]], axis=1)
    prod = act * lin_shift
    y = jax.lax.dot_general(prod.astype(jnp.bfloat16), s_ref[...], (((1,), (0,)), ((), ())),
                            preferred_element_type=jnp.float32)
    o = jax.lax.dot_general(y.astype(jnp.bfloat16), w2_ref[0].astype(jnp.bfloat16), _DN,
                            preferred_element_type=jnp.float32)
    ecol = jax.lax.broadcasted_iota(jnp.int32, (T, E), 1)
    coef = jnp.sum(jnp.where(ecol == e, c_ref[...], 0.0), axis=1, keepdims=True)
    acc = o * coef

    @pl.when(c == NCHUNK - 1)
    def _bias2():
        out_ref[...] += b2_ref[0] * coef

    out_ref[...] += acc


@functools.partial(jax.jit, static_argnames=("interpret",))
def kernel(x, norm_w, gate_w, gate_b, mlp1_w, mlp1_b, mlp2_w, mlp2_b,
           interpret=False):
    grid = (E, NCHUNK)
    half = I2 // (2 * NCHUNK)
    out = pl.pallas_call(
        _moe_body,
        grid=grid,
        in_specs=[
            pl.BlockSpec((T, D), lambda e, c: (0, 0)),
            pl.BlockSpec((1, D), lambda e, c: (0, 0)),
            pl.BlockSpec((E, D), lambda e, c: (0, 0)),
            pl.BlockSpec((1, E), lambda e, c: (0, 0)),
            pl.BlockSpec((1, CHUNK, D), lambda e, c: (e, c, 0)),
            pl.BlockSpec((1, 1, I2), lambda e, c: (e, 0, 0)),
            pl.BlockSpec((1, D, half), lambda e, c: (e, 0, c)),
            pl.BlockSpec((1, 1, D), lambda e, c: (e, 0, 0)),
        ],
        out_specs=pl.BlockSpec((T, D), lambda e, c: (0, 0)),
        out_shape=jax.ShapeDtypeStruct((T, D), jnp.float32),
        scratch_shapes=[
            pltpu.VMEM((T, D), jnp.float32),
            pltpu.VMEM((T, E), jnp.float32),
            pltpu.VMEM((CHUNK, CHUNK // 2), jnp.bfloat16),
        ],
        interpret=interpret,
    )(x, norm_w.reshape(1, D), gate_w, gate_b.reshape(1, E),
      mlp1_w, mlp1_b.reshape(E, 1, I2), mlp2_w, mlp2_b.reshape(E, 1, D))
    return out


# speedup vs baseline: 1.2753x; 1.2753x over previous
import functools

import jax
import jax.numpy as jnp
from jax.experimental import pallas as pl
from jax.experimental.pallas import tpu as pltpu

T = 64
D = 768
E = 64
I2 = 1536
CHUNK = 1536
NCHUNK = I2 // CHUNK
LIMIT = 7.0
ALPHA = 1.702

_DN = (((1,), (1,)), ((), ()))


def _moe_body(x_ref, nw_ref, gw_ref, gb_ref, w1_ref, b1_ref, w2_ref, b2_ref,
              out_ref, t_ref, c_ref, s_ref):
    e = pl.program_id(0)
    c = pl.program_id(1)

    @pl.when((e == 0) & (c == 0))
    def _init():
        si = jax.lax.broadcasted_iota(jnp.int32, (CHUNK, CHUNK // 2), 0)
        sj = jax.lax.broadcasted_iota(jnp.int32, (CHUNK, CHUNK // 2), 1)
        s_ref[...] = (si == 2 * sj).astype(jnp.bfloat16)
        xv = x_ref[...]
        ms = jnp.mean(xv * xv, axis=1, keepdims=True)
        eps = jnp.finfo(jnp.float32).eps
        t = xv * jax.lax.rsqrt(ms + eps) * nw_ref[...]
        t_ref[...] = t
        g = jax.lax.dot_general(t, gw_ref[...], _DN,
                                preferred_element_type=jnp.float32)
        g = g + gb_ref[...]
        col = jax.lax.broadcasted_iota(jnp.int32, (T, E), 1)
        m1 = jnp.max(g, axis=1, keepdims=True)
        i1 = jnp.min(jnp.where(g == m1, col, E), axis=1, keepdims=True)
        oh1 = col == i1
        gm = jnp.where(oh1, -jnp.inf, g)
        m2 = jnp.max(gm, axis=1, keepdims=True)
        i2 = jnp.min(jnp.where(gm == m2, col, E), axis=1, keepdims=True)
        oh2 = col == i2
        p1 = jax.nn.sigmoid(m1 - m2)
        c_ref[...] = jnp.where(oh1, p1, 0.0) + jnp.where(oh2, 1.0 - p1, 0.0)
        out_ref[...] = xv

    t = t_ref[...].astype(jnp.bfloat16)
    b1 = b1_ref[0, :, pl.ds(c * CHUNK, CHUNK)]
    h = jax.lax.dot_general(t, w1_ref[0].astype(jnp.bfloat16), _DN,
                            preferred_element_type=jnp.float32) + b1
    glu = jnp.minimum(h, LIMIT)
    act = glu * jax.nn.sigmoid(ALPHA * glu)
    lin = jnp.clip(h, -LIMIT, LIMIT) + 1.0
    lin_shift = jnp.concatenate([lin[:, 1:], lin[:, :1]], axis=1)
    prod = act * lin_shift
    y = jax.lax.dot_general(prod.astype(jnp.bfloat16), s_ref[...], (((1,), (0,)), ((), ())),
                            preferred_element_type=jnp.float32)
    o = jax.lax.dot_general(y.astype(jnp.bfloat16), w2_ref[0].astype(jnp.bfloat16), _DN,
                            preferred_element_type=jnp.float32)
    ecol = jax.lax.broadcasted_iota(jnp.int32, (T, E), 1)
    coef = jnp.sum(jnp.where(ecol == e, c_ref[...], 0.0), axis=1, keepdims=True)
    acc = o * coef

    @pl.when(c == NCHUNK - 1)
    def _bias2():
        out_ref[...] += b2_ref[0] * coef

    out_ref[...] += acc


@functools.partial(jax.jit, static_argnames=("interpret",))
def kernel(x, norm_w, gate_w, gate_b, mlp1_w, mlp1_b, mlp2_w, mlp2_b,
           interpret=False):
    grid = (E, NCHUNK)
    half = I2 // (2 * NCHUNK)
    out = pl.pallas_call(
        _moe_body,
        grid=grid,
        in_specs=[
            pl.BlockSpec((T, D), lambda e, c: (0, 0)),
            pl.BlockSpec((1, D), lambda e, c: (0, 0)),
            pl.BlockSpec((E, D), lambda e, c: (0, 0)),
            pl.BlockSpec((1, E), lambda e, c: (0, 0)),
            pl.BlockSpec((1, CHUNK, D), lambda e, c: (e, c, 0)),
            pl.BlockSpec((1, 1, I2), lambda e, c: (e, 0, 0)),
            pl.BlockSpec((1, D, half), lambda e, c: (e, 0, c)),
            pl.BlockSpec((1, 1, D), lambda e, c: (e, 0, 0)),
        ],
        out_specs=pl.BlockSpec((T, D), lambda e, c: (0, 0)),
        out_shape=jax.ShapeDtypeStruct((T, D), jnp.float32),
        scratch_shapes=[
            pltpu.VMEM((T, D), jnp.float32),
            pltpu.VMEM((T, E), jnp.float32),
            pltpu.VMEM((CHUNK, CHUNK // 2), jnp.bfloat16),
        ],
        interpret=interpret,
    )(x, norm_w.reshape(1, D), gate_w, gate_b.reshape(1, E),
      mlp1_w, mlp1_b.reshape(E, 1, I2), mlp2_w, mlp2_b.reshape(E, 1, D))
    return out
